# baseline (speedup 1.0000x reference)
import jax
import jax.numpy as jnp
from jax import lax
from jax.experimental import pallas as pl
from jax.experimental.pallas import tpu as pltpu
from jax.experimental.pallas import tpu_sc as plsc

DOSE_MAX = 52.0
PTV_Q = (0.99, 0.95, 0.01)
W_VDVH = 0.3
W_CDVH = 0.2

K = 8192
KR = K // 128
BW = DOSE_MAX / K
B = 2
N = 96 * 128 * 128
NCH = 12
CR = N // NCH // 128
NW = 32
NPW = N // NW
SC_CHUNK = 4096
N_SC_CHUNKS = NPW // SC_CHUNK
HSLABS = 14
HSZ = HSLABS * K


def _pass1_body(pred_ref, tgt_ref, ptv_ref, mh_ref, ml_ref,
                packed_ref, stats_ref):
    ch = pl.program_id(0)
    x = pred_ref[0, 0]
    y = tgt_ref[0, 0]
    mp = ptv_ref[0, 0].astype(jnp.int32)
    mh = mh_ref[0, 0].astype(jnp.int32)
    ml = ml_ref[0, 0].astype(jnp.int32)

    binx = jnp.clip((x * float(K)).astype(jnp.int32), 0, K - 1)
    biny = jnp.clip((y * float(K)).astype(jnp.int32), 0, K - 1)
    combo = mp + 2 * mh + 4 * ml
    packed_ref[0, 0, 0] = binx + (biny << 13) + (combo << 26)

    mpf = mp.astype(jnp.float32)
    mhf = mh.astype(jnp.float32)
    mlf = ml.astype(jnp.float32)
    sums = [
        jnp.sum(mpf), jnp.sum(mhf), jnp.sum(mlf),
        jnp.sum(mhf * x), jnp.sum(mhf * y),
        jnp.sum(mlf * x), jnp.sum(mlf * y),
        jnp.sum((x - y) * (x - y)),
    ]
    maxes = [
        jnp.max(jnp.where(mh == 1, x, -1.0)),
        jnp.max(jnp.where(mh == 1, y, -1.0)),
        jnp.max(jnp.where(ml == 1, x, -1.0)),
        jnp.max(jnp.where(ml == 1, y, -1.0)),
    ]
    iota = lax.broadcasted_iota(jnp.int32, (1, 128), 1)
    svec = jnp.zeros((1, 128), jnp.float32)
    for i, v in enumerate(sums):
        svec = jnp.where(iota == i, v, svec)
    mvec = jnp.full((1, 128), -1.0, jnp.float32)
    for i, v in enumerate(maxes):
        mvec = jnp.where(iota == i, v, mvec)

    @pl.when(ch == 0)
    def _():
        stats_ref[0, 0:1, :] = svec
        stats_ref[0, 1:2, :] = mvec

    @pl.when(ch != 0)
    def _():
        stats_ref[0, 0:1, :] = stats_ref[0, 0:1, :] + svec
        stats_ref[0, 1:2, :] = jnp.maximum(stats_ref[0, 1:2, :], mvec)


def _sc_hist_body(packed_hbm, out_hbm, buf0, buf1, hist, sem0, sem1):
    wid = lax.axis_index("s") * 2 + lax.axis_index("c")
    base = wid * NPW

    zeros16 = jnp.zeros((16,), jnp.float32)

    @plsc.parallel_loop(0, HSZ // 16, unroll=8)
    def _(i):
        hist[pl.ds(i * 16, 16)] = zeros16

    ones16 = jnp.ones((16,), jnp.float32)

    def process(buf):
        @plsc.parallel_loop(0, SC_CHUNK // 16, unroll=8)
        def _(v):
            xv = buf[pl.ds(v * 16, 16)]
            c3 = xv >> 26
            m = c3 > 0
            cm1 = jnp.maximum(c3 - 1, 0)
            ip = (cm1 << 13) + (xv & 0x1FFF)
            ig = ((cm1 + 7) << 13) + ((xv >> 13) & 0x1FFF)
            plsc.addupdate_scatter(hist, [ip], ones16, mask=m)
            plsc.addupdate_scatter(hist, [ig], ones16, mask=m)

    bufs = (buf0, buf1)
    sems = (sem0, sem1)
    descs = [None, None]
    descs[0] = pltpu.async_copy(
        packed_hbm.at[pl.ds(base, SC_CHUNK)], buf0, sem0)
    for ch in range(N_SC_CHUNKS):
        cur = ch % 2
        if ch + 1 < N_SC_CHUNKS:
            nxt = 1 - cur
            descs[nxt] = pltpu.async_copy(
                packed_hbm.at[pl.ds(base + (ch + 1) * SC_CHUNK, SC_CHUNK)],
                bufs[nxt], sems[nxt])
        descs[cur].wait()
        process(bufs[cur])

    pltpu.sync_copy(hist, out_hbm.at[wid])


def _cumflat(a, upper, lstrict):
    rowpref = jnp.dot(a, upper, preferred_element_type=jnp.float32,
                      precision=lax.Precision.HIGHEST)
    rowsum = jnp.sum(a, axis=1, keepdims=True)
    prevrows = jnp.dot(lstrict, rowsum, preferred_element_type=jnp.float32,
                       precision=lax.Precision.HIGHEST)
    return rowpref + prevrows


_PTV_SLABS = (0, 2, 4, 6)
_HEART_SLABS = (1, 2, 5, 6)
_LUNG_SLABS = (3, 4, 5, 6)


def _final_body(h0_ref, h1_ref, stats0_ref, stats1_ref, out_ref):
    ii = lax.broadcasted_iota(jnp.int32, (KR, 128), 0)
    jj = lax.broadcasted_iota(jnp.int32, (KR, 128), 1)
    iu = lax.broadcasted_iota(jnp.int32, (128, 128), 0)
    ju = lax.broadcasted_iota(jnp.int32, (128, 128), 1)
    upper = (iu <= ju).astype(jnp.float32)
    il = lax.broadcasted_iota(jnp.int32, (KR, KR), 0)
    jl = lax.broadcasted_iota(jnp.int32, (KR, KR), 1)
    lstrict = (jl < il).astype(jnp.float32)
    flat = (ii * 128 + jj).astype(jnp.float32)

    hrefs = (h0_ref, h1_ref)
    srefs = (stats0_ref, stats1_ref)
    combined = {}
    for b in range(B):
        for s in range(HSLABS):
            acc = hrefs[b][0, s]
            for t in range(1, NW):
                acc = acc + hrefs[b][t, s]
            combined[(b, s)] = acc

    def mask_hist(pat, slabs, targ):
        off = 7 if targ else 0
        acc = None
        for s in slabs:
            v = combined[(pat, s + off)]
            acc = v if acc is None else acc + v
        return acc

    def orderstat(cum, h, r):
        j = jnp.sum((cum < r).astype(jnp.float32))
        onehot = (flat == j).astype(jnp.float32)
        cnt = jnp.sum(h * onehot)
        cumbefore = jnp.sum(cum * onehot) - cnt
        return (j + (r - cumbefore) / (cnt + 1.0)) * BW

    v_vals, v_valid, c_vals, c_valid = [], [], [], []
    for b in range(B):
        sref = srefs[b]
        n_ptv = sref[0, 0, 0]
        n_h = sref[0, 0, 1]
        n_l = sref[0, 0, 2]

        ptv_p = mask_hist(b, _PTV_SLABS, False)
        ptv_g = mask_hist(b, _PTV_SLABS, True)
        heart_d = mask_hist(b, _HEART_SLABS, False) - mask_hist(b, _HEART_SLABS, True)
        lung_d = mask_hist(b, _LUNG_SLABS, False) - mask_hist(b, _LUNG_SLABS, True)

        cum_ptv_p = _cumflat(ptv_p, upper, lstrict)
        cum_ptv_g = _cumflat(ptv_g, upper, lstrict)
        w1 = jnp.sum(jnp.abs(cum_ptv_p - cum_ptv_g)) * BW
        w1 = w1 + jnp.sum(jnp.abs(_cumflat(heart_d, upper, lstrict))) * BW
        w1 = w1 + jnp.sum(jnp.abs(_cumflat(lung_d, upper, lstrict))) * BW
        den = n_ptv + n_h + n_l
        v_vals.append(w1 / jnp.maximum(den, 1.0))
        v_valid.append((den > 0).astype(jnp.float32))

        terms = []
        nf = n_ptv
        ptv_ok = n_ptv > 0
        for q in PTV_Q:
            h = jnp.float32(q) * (nf - 1.0)
            low = jnp.floor(h)
            high = jnp.ceil(h)
            hw = h - low
            lw = 1.0 - hw
            rl = jnp.clip(low, 0.0, nf - 1.0) + 1.0
            rh = jnp.clip(high, 0.0, nf - 1.0) + 1.0
            qx = (orderstat(cum_ptv_p, ptv_p, rl) * lw
                  + orderstat(cum_ptv_p, ptv_p, rh) * hw)
            qy = (orderstat(cum_ptv_g, ptv_g, rl) * lw
                  + orderstat(cum_ptv_g, ptv_g, rh) * hw)
            terms.append(jnp.where(ptv_ok, jnp.abs(qx - qy), 0.0))
        any_ok = ptv_ok
        for (n_m, s_xi, s_yi, mx_xi, mx_yi) in (
                (n_h, 3, 4, 0, 1), (n_l, 5, 6, 2, 3)):
            ok = n_m > 0
            any_ok = any_ok | ok
            dmax = jnp.abs(sref[0, 1, mx_xi]
                           - sref[0, 1, mx_yi]) * DOSE_MAX
            nf2 = jnp.maximum(n_m, 1.0)
            dmean = jnp.abs(sref[0, 0, s_xi]
                            - sref[0, 0, s_yi]) / nf2 * DOSE_MAX
            terms.append(jnp.where(ok, dmax, 0.0))
            terms.append(jnp.where(ok, dmean, 0.0))
        c_vals.append(sum(terms))
        c_valid.append(any_ok.astype(jnp.float32))

    def avg(vals, valid):
        tot = vals[0] * valid[0] + vals[1] * valid[1]
        cnt = valid[0] + valid[1]
        return jnp.where(cnt > 0, tot / jnp.maximum(cnt, 1.0), 0.0)

    v = avg(v_vals, v_valid)
    c = avg(c_vals, c_valid)
    mse = (stats0_ref[0, 0, 7] + stats1_ref[0, 0, 7]) / float(B * N)
    w0 = max(0.0, 1.0 - W_VDVH - W_CDVH)
    out_ref[0, 0] = w0 * mse + W_VDVH * v + W_CDVH * c


def _make_pass1(pat, interpret=False):
    in_spec = pl.BlockSpec((1, 1, CR, 128), lambda c: (pat, c, 0, 0))
    return pl.pallas_call(
        _pass1_body,
        grid=(NCH,),
        in_specs=[in_spec] * 5,
        out_specs=[
            pl.BlockSpec((1, 1, 1, CR, 128), lambda c: (0, 0, c, 0, 0)),
            pl.BlockSpec((1, 2, 128), lambda c: (0, 0, 0)),
        ],
        out_shape=[
            jax.ShapeDtypeStruct((1, 1, NCH, CR, 128), jnp.int32),
            jax.ShapeDtypeStruct((1, 2, 128), jnp.float32),
        ],
        interpret=interpret,
    )


def _make_final(interpret=False):
    return pl.pallas_call(
        _final_body,
        in_specs=[pl.BlockSpec(memory_space=pltpu.VMEM)] * 4,
        out_specs=pl.BlockSpec(memory_space=pltpu.SMEM),
        out_shape=jax.ShapeDtypeStruct((1, 1), jnp.float32),
        interpret=interpret,
    )


def _make_sc_hist():
    mesh = plsc.VectorSubcoreMesh(core_axis_name="c", subcore_axis_name="s")
    return pl.kernel(
        _sc_hist_body,
        mesh=mesh,
        compiler_params=pltpu.CompilerParams(needs_layout_passes=False),
        out_type=jax.ShapeDtypeStruct((NW, HSZ), jnp.float32),
        scratch_types=[
            pltpu.VMEM((SC_CHUNK,), jnp.int32),
            pltpu.VMEM((SC_CHUNK,), jnp.int32),
            pltpu.VMEM((HSZ,), jnp.float32),
            pltpu.SemaphoreType.DMA,
            pltpu.SemaphoreType.DMA,
        ],
    )


def kernel(pred, target, ptv_mask, oar_mask_heart, oar_mask_lung):
    shp = (B, NCH, CR, 128)
    p = pred.astype(jnp.float32).reshape(shp)
    g = target.astype(jnp.float32).reshape(shp)
    mp = ptv_mask.astype(jnp.int8).reshape(shp)
    mh = oar_mask_heart.astype(jnp.int8).reshape(shp)
    ml = oar_mask_lung.astype(jnp.int8).reshape(shp)

    sc = _make_sc_hist()
    ths, stats = [], []
    for b in range(B):
        packed_b, stats_b = _make_pass1(b)(p, g, mp, mh, ml)
        ths.append(sc(packed_b.reshape(N)).reshape(NW, HSLABS, KR, 128))
        stats.append(stats_b)
    out = _make_final()(ths[0], ths[1], stats[0], stats[1])
    return out[0, 0]


# speedup vs baseline: 1.1615x; 1.1615x over previous
import jax
import jax.numpy as jnp
from jax import lax
from jax.experimental import pallas as pl
from jax.experimental.pallas import tpu as pltpu
from jax.experimental.pallas import tpu_sc as plsc

DOSE_MAX = 52.0
PTV_Q = (0.99, 0.95, 0.01)
W_VDVH = 0.3
W_CDVH = 0.2

K = 8192
KR = K // 128
BW = DOSE_MAX / K
B = 2
N = 96 * 128 * 128
NCH = 12
CR = N // NCH // 128
NW = 32
WPP = 16
NPW = N // WPP
SC_CHUNK = 4096
N_SC_CHUNKS = NPW // SC_CHUNK
HSLABS = 14
HSZ = HSLABS * K


def _pass1_body(pred_ref, tgt_ref, ptv_ref, mh_ref, ml_ref,
                packed_ref, stats_ref):
    ch = pl.program_id(1)
    x = pred_ref[0, 0]
    y = tgt_ref[0, 0]
    mp = ptv_ref[0, 0].astype(jnp.int32)
    mh = mh_ref[0, 0].astype(jnp.int32)
    ml = ml_ref[0, 0].astype(jnp.int32)

    binx = jnp.clip((x * float(K)).astype(jnp.int32), 0, K - 1)
    biny = jnp.clip((y * float(K)).astype(jnp.int32), 0, K - 1)
    combo = mp + 2 * mh + 4 * ml
    packed_ref[0, 0, 0] = binx + (biny << 13) + (combo << 26)

    mpf = mp.astype(jnp.float32)
    mhf = mh.astype(jnp.float32)
    mlf = ml.astype(jnp.float32)
    sums = [
        jnp.sum(mpf), jnp.sum(mhf), jnp.sum(mlf),
        jnp.sum(mhf * x), jnp.sum(mhf * y),
        jnp.sum(mlf * x), jnp.sum(mlf * y),
        jnp.sum((x - y) * (x - y)),
    ]
    maxes = [
        jnp.max(jnp.where(mh == 1, x, -1.0)),
        jnp.max(jnp.where(mh == 1, y, -1.0)),
        jnp.max(jnp.where(ml == 1, x, -1.0)),
        jnp.max(jnp.where(ml == 1, y, -1.0)),
    ]
    iota = lax.broadcasted_iota(jnp.int32, (1, 128), 1)
    svec = jnp.zeros((1, 128), jnp.float32)
    for i, v in enumerate(sums):
        svec = jnp.where(iota == i, v, svec)
    mvec = jnp.full((1, 128), -1.0, jnp.float32)
    for i, v in enumerate(maxes):
        mvec = jnp.where(iota == i, v, mvec)

    @pl.when(ch == 0)
    def _():
        stats_ref[0, 0:1, :] = svec
        stats_ref[0, 1:2, :] = mvec

    @pl.when(ch != 0)
    def _():
        stats_ref[0, 0:1, :] = stats_ref[0, 0:1, :] + svec
        stats_ref[0, 1:2, :] = jnp.maximum(stats_ref[0, 1:2, :], mvec)


def _sc_hist_body(packed_hbm, out_hbm, buf0, buf1, hist, sem0, sem1):
    wid = lax.axis_index("s") * 2 + lax.axis_index("c")
    base = (wid // WPP) * N + (wid % WPP) * NPW

    zeros16 = jnp.zeros((16,), jnp.float32)

    @plsc.parallel_loop(0, HSZ // 16, unroll=8)
    def _(i):
        hist[pl.ds(i * 16, 16)] = zeros16

    ones16 = jnp.ones((16,), jnp.float32)

    def process(buf):
        @plsc.parallel_loop(0, SC_CHUNK // 16, unroll=8)
        def _(v):
            xv = buf[pl.ds(v * 16, 16)]
            c3 = xv >> 26
            m = c3 > 0
            cm1 = jnp.maximum(c3 - 1, 0)
            ip = (cm1 << 13) + (xv & 0x1FFF)
            ig = ((cm1 + 7) << 13) + ((xv >> 13) & 0x1FFF)
            plsc.addupdate_scatter(hist, [ip], ones16, mask=m)
            plsc.addupdate_scatter(hist, [ig], ones16, mask=m)

    bufs = (buf0, buf1)
    sems = (sem0, sem1)
    descs = [None, None]
    descs[0] = pltpu.async_copy(
        packed_hbm.at[pl.ds(base, SC_CHUNK)], buf0, sem0)
    for ch in range(N_SC_CHUNKS):
        cur = ch % 2
        if ch + 1 < N_SC_CHUNKS:
            nxt = 1 - cur
            descs[nxt] = pltpu.async_copy(
                packed_hbm.at[pl.ds(base + (ch + 1) * SC_CHUNK, SC_CHUNK)],
                bufs[nxt], sems[nxt])
        descs[cur].wait()
        process(bufs[cur])

    pltpu.sync_copy(hist, out_hbm.at[wid])


def _cumflat(a, upper, lstrict):
    rowpref = jnp.dot(a, upper, preferred_element_type=jnp.float32,
                      precision=lax.Precision.HIGHEST)
    rowsum = jnp.sum(a, axis=1, keepdims=True)
    prevrows = jnp.dot(lstrict, rowsum, preferred_element_type=jnp.float32,
                       precision=lax.Precision.HIGHEST)
    return rowpref + prevrows


_PTV_SLABS = (0, 2, 4, 6)
_HEART_SLABS = (1, 2, 5, 6)
_LUNG_SLABS = (3, 4, 5, 6)


def _final_body(hists_ref, stats_ref, out_ref):
    ii = lax.broadcasted_iota(jnp.int32, (KR, 128), 0)
    jj = lax.broadcasted_iota(jnp.int32, (KR, 128), 1)
    iu = lax.broadcasted_iota(jnp.int32, (128, 128), 0)
    ju = lax.broadcasted_iota(jnp.int32, (128, 128), 1)
    upper = (iu <= ju).astype(jnp.float32)
    il = lax.broadcasted_iota(jnp.int32, (KR, KR), 0)
    jl = lax.broadcasted_iota(jnp.int32, (KR, KR), 1)
    lstrict = (jl < il).astype(jnp.float32)
    flat = (ii * 128 + jj).astype(jnp.float32)

    def mask_hist(pat, slabs, targ):
        off = 7 if targ else 0
        acc = None
        for t in range(WPP):
            for s in slabs:
                row = ((WPP * pat + t) * HSLABS + s + off) * KR
                v = hists_ref[row:row + KR, :]
                acc = v if acc is None else acc + v
        return acc

    def orderstat(cum, h, r):
        j = jnp.sum((cum < r).astype(jnp.float32))
        onehot = (flat == j).astype(jnp.float32)
        cnt = jnp.sum(h * onehot)
        cumbefore = jnp.sum(cum * onehot) - cnt
        return (j + (r - cumbefore) / (cnt + 1.0)) * BW

    v_vals, v_valid, c_vals, c_valid = [], [], [], []
    for b in range(B):
        n_ptv = stats_ref[b, 0, 0]
        n_h = stats_ref[b, 0, 1]
        n_l = stats_ref[b, 0, 2]

        ptv_p = mask_hist(b, _PTV_SLABS, False)
        ptv_g = mask_hist(b, _PTV_SLABS, True)
        heart_d = mask_hist(b, _HEART_SLABS, False) - mask_hist(b, _HEART_SLABS, True)
        lung_d = mask_hist(b, _LUNG_SLABS, False) - mask_hist(b, _LUNG_SLABS, True)

        cum_ptv_p = _cumflat(ptv_p, upper, lstrict)
        cum_ptv_g = _cumflat(ptv_g, upper, lstrict)
        w1 = jnp.sum(jnp.abs(cum_ptv_p - cum_ptv_g)) * BW
        w1 = w1 + jnp.sum(jnp.abs(_cumflat(heart_d, upper, lstrict))) * BW
        w1 = w1 + jnp.sum(jnp.abs(_cumflat(lung_d, upper, lstrict))) * BW
        den = n_ptv + n_h + n_l
        v_vals.append(w1 / jnp.maximum(den, 1.0))
        v_valid.append((den > 0).astype(jnp.float32))

        terms = []
        nf = n_ptv
        ptv_ok = n_ptv > 0
        for q in PTV_Q:
            h = jnp.float32(q) * (nf - 1.0)
            low = jnp.floor(h)
            high = jnp.ceil(h)
            hw = h - low
            lw = 1.0 - hw
            rl = jnp.clip(low, 0.0, nf - 1.0) + 1.0
            rh = jnp.clip(high, 0.0, nf - 1.0) + 1.0
            qx = (orderstat(cum_ptv_p, ptv_p, rl) * lw
                  + orderstat(cum_ptv_p, ptv_p, rh) * hw)
            qy = (orderstat(cum_ptv_g, ptv_g, rl) * lw
                  + orderstat(cum_ptv_g, ptv_g, rh) * hw)
            terms.append(jnp.where(ptv_ok, jnp.abs(qx - qy), 0.0))
        any_ok = ptv_ok
        for (n_m, s_xi, s_yi, mx_xi, mx_yi) in (
                (n_h, 3, 4, 0, 1), (n_l, 5, 6, 2, 3)):
            ok = n_m > 0
            any_ok = any_ok | ok
            dmax = jnp.abs(stats_ref[b, 1, mx_xi]
                           - stats_ref[b, 1, mx_yi]) * DOSE_MAX
            nf2 = jnp.maximum(n_m, 1.0)
            dmean = jnp.abs(stats_ref[b, 0, s_xi]
                            - stats_ref[b, 0, s_yi]) / nf2 * DOSE_MAX
            terms.append(jnp.where(ok, dmax, 0.0))
            terms.append(jnp.where(ok, dmean, 0.0))
        c_vals.append(sum(terms))
        c_valid.append(any_ok.astype(jnp.float32))

    def avg(vals, valid):
        tot = vals[0] * valid[0] + vals[1] * valid[1]
        cnt = valid[0] + valid[1]
        return jnp.where(cnt > 0, tot / jnp.maximum(cnt, 1.0), 0.0)

    v = avg(v_vals, v_valid)
    c = avg(c_vals, c_valid)
    mse = (stats_ref[0, 0, 7] + stats_ref[1, 0, 7]) / float(B * N)
    w0 = max(0.0, 1.0 - W_VDVH - W_CDVH)
    out_ref[0, 0] = w0 * mse + W_VDVH * v + W_CDVH * c


def _make_pass1(interpret=False):
    in_spec = pl.BlockSpec((1, 1, CR, 128), lambda p, c: (p, c, 0, 0))
    return pl.pallas_call(
        _pass1_body,
        grid=(B, NCH),
        in_specs=[in_spec] * 5,
        out_specs=[
            pl.BlockSpec((1, 1, 1, CR, 128), lambda p, c: (p, 0, c, 0, 0)),
            pl.BlockSpec((1, 2, 128), lambda p, c: (p, 0, 0)),
        ],
        out_shape=[
            jax.ShapeDtypeStruct((B, 1, NCH, CR, 128), jnp.int32),
            jax.ShapeDtypeStruct((B, 2, 128), jnp.float32),
        ],
        interpret=interpret,
    )


def _make_final(interpret=False):
    return pl.pallas_call(
        _final_body,
        in_specs=[
            pl.BlockSpec(memory_space=pltpu.VMEM),
            pl.BlockSpec(memory_space=pltpu.VMEM),
        ],
        out_specs=pl.BlockSpec(memory_space=pltpu.SMEM),
        out_shape=jax.ShapeDtypeStruct((1, 1), jnp.float32),
        interpret=interpret,
    )


def _make_sc_hist():
    mesh = plsc.VectorSubcoreMesh(core_axis_name="c", subcore_axis_name="s")
    return pl.kernel(
        _sc_hist_body,
        mesh=mesh,
        compiler_params=pltpu.CompilerParams(needs_layout_passes=False),
        out_type=jax.ShapeDtypeStruct((NW, HSZ), jnp.float32),
        scratch_types=[
            pltpu.VMEM((SC_CHUNK,), jnp.int32),
            pltpu.VMEM((SC_CHUNK,), jnp.int32),
            pltpu.VMEM((HSZ,), jnp.float32),
            pltpu.SemaphoreType.DMA,
            pltpu.SemaphoreType.DMA,
        ],
    )


def kernel(pred, target, ptv_mask, oar_mask_heart, oar_mask_lung):
    shp = (B, NCH, CR, 128)
    p = pred.astype(jnp.float32).reshape(shp)
    g = target.astype(jnp.float32).reshape(shp)
    mp = ptv_mask.astype(jnp.int8).reshape(shp)
    mh = oar_mask_heart.astype(jnp.int8).reshape(shp)
    ml = oar_mask_lung.astype(jnp.int8).reshape(shp)

    packed, stats = _make_pass1()(p, g, mp, mh, ml)
    tile_hists = _make_sc_hist()(packed.reshape(B * N))
    out = _make_final()(tile_hists.reshape(NW * HSLABS * KR, 128), stats)
    return out[0, 0]


# speedup vs baseline: 1.1792x; 1.0153x over previous
import jax
import jax.numpy as jnp
from jax import lax
from jax.experimental import pallas as pl
from jax.experimental.pallas import tpu as pltpu
from jax.experimental.pallas import tpu_sc as plsc

DOSE_MAX = 52.0
PTV_Q = (0.99, 0.95, 0.01)
W_VDVH = 0.3
W_CDVH = 0.2

K = 8192
KR = K // 128
BW = DOSE_MAX / K
B = 2
N = 96 * 128 * 128
NCH = 12
CR = N // NCH // 128
NW = 32
WPP = 16
NPW = N // WPP
SC_CHUNK = 6144
N_SC_CHUNKS = NPW // SC_CHUNK
HSLABS = 14
HSZ = HSLABS * K


def _pass1_body(pred_ref, tgt_ref, ptv_ref, mh_ref, ml_ref,
                packed_ref, stats_ref):
    ch = pl.program_id(1)
    x = pred_ref[0, 0]
    y = tgt_ref[0, 0]
    mp = ptv_ref[0, 0].astype(jnp.int32)
    mh = mh_ref[0, 0].astype(jnp.int32)
    ml = ml_ref[0, 0].astype(jnp.int32)

    binx = jnp.clip((x * float(K)).astype(jnp.int32), 0, K - 1)
    biny = jnp.clip((y * float(K)).astype(jnp.int32), 0, K - 1)
    combo = mp + 2 * mh + 4 * ml
    packed_ref[0, 0, 0] = binx + (biny << 13) + (combo << 26)

    mpf = mp.astype(jnp.float32)
    mhf = mh.astype(jnp.float32)
    mlf = ml.astype(jnp.float32)
    sums = [
        jnp.sum(mpf), jnp.sum(mhf), jnp.sum(mlf),
        jnp.sum(mhf * x), jnp.sum(mhf * y),
        jnp.sum(mlf * x), jnp.sum(mlf * y),
        jnp.sum((x - y) * (x - y)),
    ]
    maxes = [
        jnp.max(jnp.where(mh == 1, x, -1.0)),
        jnp.max(jnp.where(mh == 1, y, -1.0)),
        jnp.max(jnp.where(ml == 1, x, -1.0)),
        jnp.max(jnp.where(ml == 1, y, -1.0)),
    ]
    iota = lax.broadcasted_iota(jnp.int32, (1, 128), 1)
    svec = jnp.zeros((1, 128), jnp.float32)
    for i, v in enumerate(sums):
        svec = jnp.where(iota == i, v, svec)
    mvec = jnp.full((1, 128), -1.0, jnp.float32)
    for i, v in enumerate(maxes):
        mvec = jnp.where(iota == i, v, mvec)

    @pl.when(ch == 0)
    def _():
        stats_ref[0, 0:1, :] = svec
        stats_ref[0, 1:2, :] = mvec

    @pl.when(ch != 0)
    def _():
        stats_ref[0, 0:1, :] = stats_ref[0, 0:1, :] + svec
        stats_ref[0, 1:2, :] = jnp.maximum(stats_ref[0, 1:2, :], mvec)


def _sc_hist_body(packed_hbm, out_hbm, buf0, buf1, hist, sem0, sem1):
    wid = lax.axis_index("s") * 2 + lax.axis_index("c")
    base = (wid // WPP) * N + (wid % WPP) * NPW

    zeros16 = jnp.zeros((16,), jnp.float32)

    @plsc.parallel_loop(0, HSZ // 16, unroll=16)
    def _(i):
        hist[pl.ds(i * 16, 16)] = zeros16

    ones16 = jnp.ones((16,), jnp.float32)

    def process(buf):
        @plsc.parallel_loop(0, SC_CHUNK // 16, unroll=12)
        def _(v):
            xv = buf[pl.ds(v * 16, 16)]
            c3 = xv >> 26
            m = c3 > 0
            cm1 = jnp.maximum(c3 - 1, 0)
            ip = (cm1 << 13) + (xv & 0x1FFF)
            ig = ((cm1 + 7) << 13) + ((xv >> 13) & 0x1FFF)
            plsc.addupdate_scatter(hist, [ip], ones16, mask=m)
            plsc.addupdate_scatter(hist, [ig], ones16, mask=m)

    bufs = (buf0, buf1)
    sems = (sem0, sem1)
    descs = [None, None]
    descs[0] = pltpu.async_copy(
        packed_hbm.at[pl.ds(base, SC_CHUNK)], buf0, sem0)
    for ch in range(N_SC_CHUNKS):
        cur = ch % 2
        if ch + 1 < N_SC_CHUNKS:
            nxt = 1 - cur
            descs[nxt] = pltpu.async_copy(
                packed_hbm.at[pl.ds(base + (ch + 1) * SC_CHUNK, SC_CHUNK)],
                bufs[nxt], sems[nxt])
        descs[cur].wait()
        process(bufs[cur])

    pltpu.sync_copy(hist, out_hbm.at[wid])


def _cumflat(a, upper, lstrict):
    rowpref = jnp.dot(a, upper, preferred_element_type=jnp.float32,
                      precision=lax.Precision.HIGHEST)
    rowsum = jnp.sum(a, axis=1, keepdims=True)
    prevrows = jnp.dot(lstrict, rowsum, preferred_element_type=jnp.float32,
                       precision=lax.Precision.HIGHEST)
    return rowpref + prevrows


_PTV_SLABS = (0, 2, 4, 6)
_HEART_SLABS = (1, 2, 5, 6)
_LUNG_SLABS = (3, 4, 5, 6)


def _final_body(hists_ref, stats_ref, out_ref):
    ii = lax.broadcasted_iota(jnp.int32, (KR, 128), 0)
    jj = lax.broadcasted_iota(jnp.int32, (KR, 128), 1)
    iu = lax.broadcasted_iota(jnp.int32, (128, 128), 0)
    ju = lax.broadcasted_iota(jnp.int32, (128, 128), 1)
    upper = (iu <= ju).astype(jnp.float32)
    il = lax.broadcasted_iota(jnp.int32, (KR, KR), 0)
    jl = lax.broadcasted_iota(jnp.int32, (KR, KR), 1)
    lstrict = (jl < il).astype(jnp.float32)
    flat = (ii * 128 + jj).astype(jnp.float32)

    def mask_hist(pat, slabs, targ):
        off = 7 if targ else 0
        acc = None
        for t in range(WPP):
            for s in slabs:
                row = ((WPP * pat + t) * HSLABS + s + off) * KR
                v = hists_ref[row:row + KR, :]
                acc = v if acc is None else acc + v
        return acc

    def orderstat(cum, h, r):
        j = jnp.sum((cum < r).astype(jnp.float32))
        onehot = (flat == j).astype(jnp.float32)
        cnt = jnp.sum(h * onehot)
        cumbefore = jnp.sum(cum * onehot) - cnt
        return (j + (r - cumbefore) / (cnt + 1.0)) * BW

    v_vals, v_valid, c_vals, c_valid = [], [], [], []
    for b in range(B):
        n_ptv = stats_ref[b, 0, 0]
        n_h = stats_ref[b, 0, 1]
        n_l = stats_ref[b, 0, 2]

        ptv_p = mask_hist(b, _PTV_SLABS, False)
        ptv_g = mask_hist(b, _PTV_SLABS, True)
        heart_d = mask_hist(b, _HEART_SLABS, False) - mask_hist(b, _HEART_SLABS, True)
        lung_d = mask_hist(b, _LUNG_SLABS, False) - mask_hist(b, _LUNG_SLABS, True)

        cum_ptv_p = _cumflat(ptv_p, upper, lstrict)
        cum_ptv_g = _cumflat(ptv_g, upper, lstrict)
        w1 = jnp.sum(jnp.abs(cum_ptv_p - cum_ptv_g)) * BW
        w1 = w1 + jnp.sum(jnp.abs(_cumflat(heart_d, upper, lstrict))) * BW
        w1 = w1 + jnp.sum(jnp.abs(_cumflat(lung_d, upper, lstrict))) * BW
        den = n_ptv + n_h + n_l
        v_vals.append(w1 / jnp.maximum(den, 1.0))
        v_valid.append((den > 0).astype(jnp.float32))

        terms = []
        nf = n_ptv
        ptv_ok = n_ptv > 0
        for q in PTV_Q:
            h = jnp.float32(q) * (nf - 1.0)
            low = jnp.floor(h)
            high = jnp.ceil(h)
            hw = h - low
            lw = 1.0 - hw
            rl = jnp.clip(low, 0.0, nf - 1.0) + 1.0
            rh = jnp.clip(high, 0.0, nf - 1.0) + 1.0
            qx = (orderstat(cum_ptv_p, ptv_p, rl) * lw
                  + orderstat(cum_ptv_p, ptv_p, rh) * hw)
            qy = (orderstat(cum_ptv_g, ptv_g, rl) * lw
                  + orderstat(cum_ptv_g, ptv_g, rh) * hw)
            terms.append(jnp.where(ptv_ok, jnp.abs(qx - qy), 0.0))
        any_ok = ptv_ok
        for (n_m, s_xi, s_yi, mx_xi, mx_yi) in (
                (n_h, 3, 4, 0, 1), (n_l, 5, 6, 2, 3)):
            ok = n_m > 0
            any_ok = any_ok | ok
            dmax = jnp.abs(stats_ref[b, 1, mx_xi]
                           - stats_ref[b, 1, mx_yi]) * DOSE_MAX
            nf2 = jnp.maximum(n_m, 1.0)
            dmean = jnp.abs(stats_ref[b, 0, s_xi]
                            - stats_ref[b, 0, s_yi]) / nf2 * DOSE_MAX
            terms.append(jnp.where(ok, dmax, 0.0))
            terms.append(jnp.where(ok, dmean, 0.0))
        c_vals.append(sum(terms))
        c_valid.append(any_ok.astype(jnp.float32))

    def avg(vals, valid):
        tot = vals[0] * valid[0] + vals[1] * valid[1]
        cnt = valid[0] + valid[1]
        return jnp.where(cnt > 0, tot / jnp.maximum(cnt, 1.0), 0.0)

    v = avg(v_vals, v_valid)
    c = avg(c_vals, c_valid)
    mse = (stats_ref[0, 0, 7] + stats_ref[1, 0, 7]) / float(B * N)
    w0 = max(0.0, 1.0 - W_VDVH - W_CDVH)
    out_ref[0, 0] = w0 * mse + W_VDVH * v + W_CDVH * c


def _make_pass1(interpret=False):
    in_spec = pl.BlockSpec((1, 1, CR, 128), lambda p, c: (p, c, 0, 0))
    return pl.pallas_call(
        _pass1_body,
        grid=(B, NCH),
        in_specs=[in_spec] * 5,
        out_specs=[
            pl.BlockSpec((1, 1, 1, CR, 128), lambda p, c: (p, 0, c, 0, 0)),
            pl.BlockSpec((1, 2, 128), lambda p, c: (p, 0, 0)),
        ],
        out_shape=[
            jax.ShapeDtypeStruct((B, 1, NCH, CR, 128), jnp.int32),
            jax.ShapeDtypeStruct((B, 2, 128), jnp.float32),
        ],
        interpret=interpret,
    )


def _make_final(interpret=False):
    return pl.pallas_call(
        _final_body,
        in_specs=[
            pl.BlockSpec(memory_space=pltpu.VMEM),
            pl.BlockSpec(memory_space=pltpu.VMEM),
        ],
        out_specs=pl.BlockSpec(memory_space=pltpu.SMEM),
        out_shape=jax.ShapeDtypeStruct((1, 1), jnp.float32),
        interpret=interpret,
    )


def _make_sc_hist():
    mesh = plsc.VectorSubcoreMesh(core_axis_name="c", subcore_axis_name="s")
    return pl.kernel(
        _sc_hist_body,
        mesh=mesh,
        compiler_params=pltpu.CompilerParams(needs_layout_passes=False),
        out_type=jax.ShapeDtypeStruct((NW, HSZ), jnp.float32),
        scratch_types=[
            pltpu.VMEM((SC_CHUNK,), jnp.int32),
            pltpu.VMEM((SC_CHUNK,), jnp.int32),
            pltpu.VMEM((HSZ,), jnp.float32),
            pltpu.SemaphoreType.DMA,
            pltpu.SemaphoreType.DMA,
        ],
    )


def kernel(pred, target, ptv_mask, oar_mask_heart, oar_mask_lung):
    shp = (B, NCH, CR, 128)
    p = pred.astype(jnp.float32).reshape(shp)
    g = target.astype(jnp.float32).reshape(shp)
    mp = ptv_mask.astype(jnp.int8).reshape(shp)
    mh = oar_mask_heart.astype(jnp.int8).reshape(shp)
    ml = oar_mask_lung.astype(jnp.int8).reshape(shp)

    packed, stats = _make_pass1()(p, g, mp, mh, ml)
    tile_hists = _make_sc_hist()(packed.reshape(B * N))
    out = _make_final()(tile_hists.reshape(NW * HSLABS * KR, 128), stats)
    return out[0, 0]
